# baseline (speedup 1.0000x reference)
import jax
import jax.numpy as jnp
from jax import lax
from jax.experimental import pallas as pl
from jax.experimental.pallas import tpu as pltpu

B = 4
SEQ = 8192
D_MODEL = 4096
HIDDEN = 4096
NUM_LAYERS = 16

C_ROWS = 128
NBUF = 8
CPB = SEQ // C_ROWS
NCHUNK = B * CPB

H_BLK = 512
NSUB = 4
SUB = H_BLK // NSUB
NH = HIDDEN // H_BLK
GRID = 1 + NH


def _router_kernel(x_hbm,
                   w1a_ref, w1b_ref, w1c_ref, w1d_ref,
                   w2a_ref, w2b_ref, w2c_ref, w2d_ref,
                   b1_ref, b2_ref, w3_ref, b3_ref,
                   logits_ref, idx_ref,
                   bufs_ref, acc_ref, xp_ref, h2_ref, sems):
    w1_refs = (w1a_ref, w1b_ref, w1c_ref, w1d_ref)
    w2_refs = (w2a_ref, w2b_ref, w2c_ref, w2d_ref)
    i = pl.program_id(0)

    def _chunk_copy(c, slot):
        return pltpu.make_async_copy(
            x_hbm.at[pl.ds(c * C_ROWS, C_ROWS), :],
            bufs_ref.at[slot],
            sems.at[slot])

    @pl.when(i == 0)
    def _pool():
        for k in range(NBUF):
            _chunk_copy(k, k).start()
        for b in range(B):
            def body(t, acc):
                gc = b * CPB + t
                slot = lax.rem(gc, NBUF)
                _chunk_copy(gc, slot).wait()
                acc = acc + jnp.sum(bufs_ref[slot], axis=0, keepdims=True)
                nxt = gc + NBUF

                @pl.when(nxt < NCHUNK)
                def _refill():
                    _chunk_copy(nxt, slot).start()

                return acc

            accb = lax.fori_loop(
                0, CPB, body, jnp.zeros((1, D_MODEL), jnp.float32))
            acc_ref[b:b + 1, :] = accb

    @pl.when(i >= 1)
    def _mlp():
        j = i - 1

        @pl.when(j == 0)
        def _prep():
            xp_ref[...] = acc_ref[...] * (1.0 / SEQ)

        xp = xp_ref[...]
        part = None
        for k in range(NSUB):
            pre1 = lax.dot_general(xp, w1_refs[k][...],
                                   (((1,), (1,)), ((), ())),
                                   preferred_element_type=jnp.float32)
            h1 = jax.nn.gelu(pre1 + b1_ref[0, :, k * SUB:(k + 1) * SUB])
            p = lax.dot_general(h1, w2_refs[k][...],
                                (((1,), (1,)), ((), ())),
                                preferred_element_type=jnp.float32)
            part = p if part is None else part + p

        @pl.when(j == 0)
        def _set():
            h2_ref[...] = part

        @pl.when(j > 0)
        def _add():
            h2_ref[...] += part

        @pl.when(j == NH - 1)
        def _final():
            h2 = jax.nn.gelu(h2_ref[...] + b2_ref[...])
            logits = lax.dot_general(h2, w3_ref[...],
                                     (((1,), (1,)), ((), ())),
                                     preferred_element_type=jnp.float32)
            logits = logits + b3_ref[...]
            logits_ref[...] = logits
            col = lax.broadcasted_iota(jnp.int32, (B, NUM_LAYERS), 1)
            maxv = jnp.max(logits, axis=1, keepdims=True)
            idx_ref[...] = jnp.min(
                jnp.where(logits == maxv, col, NUM_LAYERS),
                axis=1, keepdims=True)


def _w1_spec(k):
    return pl.BlockSpec(
        (SUB, D_MODEL),
        lambda i, k=k: (NSUB * jnp.clip(i - 1, 0, NH - 1) + k, 0))


def _w2_spec(k):
    return pl.BlockSpec(
        (HIDDEN, SUB),
        lambda i, k=k: (0, NSUB * jnp.clip(i - 1, 0, NH - 1) + k))


def kernel(x, W1, b1, W2, b2, W3, b3):
    x2 = x.reshape(B * SEQ, D_MODEL)
    b1r = b1.reshape(NH, 1, H_BLK)
    b2r = b2.reshape(1, HIDDEN)
    b3r = b3.reshape(1, NUM_LAYERS)

    logits, idx = pl.pallas_call(
        _router_kernel,
        grid=(GRID,),
        in_specs=(
            [pl.BlockSpec(memory_space=pl.ANY)]
            + [_w1_spec(k) for k in range(NSUB)]
            + [_w2_spec(k) for k in range(NSUB)]
            + [pl.BlockSpec((1, 1, H_BLK),
                            lambda i: (jnp.clip(i - 1, 0, NH - 1), 0, 0)),
               pl.BlockSpec((1, HIDDEN), lambda i: (0, 0)),
               pl.BlockSpec((NUM_LAYERS, HIDDEN), lambda i: (0, 0)),
               pl.BlockSpec((1, NUM_LAYERS), lambda i: (0, 0))]
        ),
        out_specs=[
            pl.BlockSpec((B, NUM_LAYERS), lambda i: (0, 0)),
            pl.BlockSpec((B, 1), lambda i: (0, 0)),
        ],
        out_shape=[
            jax.ShapeDtypeStruct((B, NUM_LAYERS), jnp.float32),
            jax.ShapeDtypeStruct((B, 1), jnp.int32),
        ],
        scratch_shapes=[
            pltpu.VMEM((NBUF, C_ROWS, D_MODEL), jnp.float32),
            pltpu.VMEM((B, D_MODEL), jnp.float32),
            pltpu.VMEM((B, D_MODEL), jnp.float32),
            pltpu.VMEM((B, HIDDEN), jnp.float32),
            pltpu.SemaphoreType.DMA((NBUF,)),
        ],
        compiler_params=pltpu.CompilerParams(
            dimension_semantics=("arbitrary",)),
    )(x2, W1, W1, W1, W1, W2, W2, W2, W2, b1r, b2r, W3, b3r)

    return (idx.reshape(B), logits)
